# baseline (speedup 1.0000x reference)
import functools

import jax
import jax.numpy as jnp
from jax import lax
from jax.experimental import pallas as pl
from jax.experimental.pallas import tpu as pltpu
from jax.experimental.pallas import tpu_sc as plsc

N = 10000
E = 160000
NVT = 6
VHS = 100
H = VHS + NVT
NL = 4
NROUNDS = 2

UW = 128
NP = 10240
ROWS_PER_SUB = NP // 16
NWORK = 32
CH = 128
EROWS = 1328
EPAD = EROWS * 128
BN = 1000
GRID = N // BN

_C11 = (((1,), (1,)), ((), ()))


def _dot_t(a, b):
    return lax.dot_general(a, b, _C11, precision=lax.Precision.HIGHEST,
                           preferred_element_type=jnp.float32)


def _u_rows(hs1, gw, gb, mw):
    g = jax.nn.sigmoid(_dot_t(hs1, gw) + gb)
    p = jnp.tanh(_dot_t(hs1, mw))
    u = g * p
    rows = u.shape[0]
    return jnp.concatenate(
        [u, jnp.ones((rows, 1), jnp.float32),
         jnp.zeros((rows, UW - VHS - 1), jnp.float32)], axis=1)


_SC_MESH = plsc.VectorSubcoreMesh(core_axis_name="c", subcore_axis_name="s")


NBUF = 2
KMAX = 40


@functools.partial(
    pl.kernel,
    mesh=_SC_MESH,
    out_type=jax.ShapeDtypeStruct((2 * NP, UW), jnp.float32),
    scratch_types=[
        pltpu.VMEM((16,), jnp.int32),
        pltpu.VMEM((KMAX, CH), jnp.int32),
        pltpu.VMEM((KMAX, CH), jnp.int32),
        pltpu.VMEM((NBUF, CH, UW), jnp.float32),
        pltpu.VMEM_SHARED((NP, UW), jnp.float32),
        [pltpu.SemaphoreType.DMA] * NBUF,
        [pltpu.SemaphoreType.DMA] * NBUF,
    ],
)
def _edge_aggregate_sc(u_hbm, src_hbm, dst_hbm, params_hbm, zeros_hbm, out_hbm,
                       params_v, src_iv, dst_iv, rows_v, ps_sh, gsem, ssem):
    c = lax.axis_index("c")
    s = lax.axis_index("s")
    w = c * 16 + s

    pltpu.sync_copy(params_hbm, params_v)
    pv = params_v[...]

    sc0 = pv[0]
    nsup = pv[1]
    a0 = pv[2]
    c_ch = pv[3]

    zc = (c_ch - s + 15) // 16
    def zbody(k, carry):
        r0 = pl.multiple_of(a0 + (s + 16 * k) * 128, 128)
        pltpu.sync_copy(zeros_hbm, ps_sh.at[pl.ds(r0, 128)])
        return carry
    lax.fori_loop(0, zc, zbody, 0)

    k0 = pl.multiple_of(sc0 + 8 * ((nsup * w) // NWORK), 8)
    k1 = sc0 + 8 * ((nsup * (w + 1)) // NWORK)
    nw = k1 - k0
    pltpu.sync_copy(src_hbm.at[pl.ds(k0, KMAX)], src_iv)
    pltpu.sync_copy(dst_hbm.at[pl.ds(k0, KMAX)], dst_iv)
    plsc.subcore_barrier()

    def fire_gather(jj, b):
        pltpu.async_copy(u_hbm.at[src_iv.at[jj]], rows_v.at[b], gsem[b])

    def wait_gather(b):
        pltpu.make_async_copy(zeros_hbm, rows_v.at[b], gsem[b]).wait()

    def fire_scatter(jj, b):
        pltpu.async_copy(rows_v.at[b], ps_sh.at[dst_iv.at[jj]], ssem[b],
                         add=True)

    def wait_scatter(b):
        pltpu.make_async_copy(zeros_hbm, rows_v.at[b], ssem[b]).wait()

    n_grp = (nw + NBUF - 1) // NBUF
    def group(i, carry):
        for b in range(NBUF):
            jj = i * NBUF + b
            @pl.when((jj < nw) & (i > 0))
            def _():
                wait_scatter(b)
            @pl.when(jj < nw)
            def _():
                fire_gather(jj, b)
        for b in range(NBUF):
            jj = i * NBUF + b
            @pl.when(jj < nw)
            def _():
                wait_gather(b)
                fire_scatter(jj, b)
        return carry
    lax.fori_loop(0, n_grp, group, 0)
    for b in range(NBUF):
        @pl.when(b < nw)
        def _():
            wait_scatter(b)
    plsc.subcore_barrier()

    def obody(k, carry):
        r0 = pl.multiple_of(a0 + (s + 16 * k) * 128, 128)
        pltpu.sync_copy(ps_sh.at[pl.ds(r0, 128)],
                        out_hbm.at[pl.ds(c * NP + r0, 128)])
        return carry
    lax.fori_loop(0, zc, obody, 0)


def _edge_aggregate(u, src_s, dst_s, params, zeros128):
    return _edge_aggregate_sc(u, src_s, dst_s, params, zeros128).reshape(
        2, NP, UW)


def _full(shape):
    return pl.BlockSpec(shape, lambda i: tuple(0 for _ in shape))


def _u_init_body(h_ref, x_ref, gw_ref, gb_ref, mw_ref, u_ref):
    hs1 = jnp.concatenate([h_ref[...], x_ref[...]], axis=1)
    u_ref[...] = _u_rows(hs1, gw_ref[...], gb_ref[...], mw_ref[...])


_u_init = pl.pallas_call(
    _u_init_body,
    grid=(GRID,),
    in_specs=[
        pl.BlockSpec((BN, VHS), lambda i: (i, 0)),
        pl.BlockSpec((BN, NVT), lambda i: (i, 0)),
        _full((VHS, H)),
        _full((1, VHS)),
        _full((VHS, H)),
    ],
    out_specs=pl.BlockSpec((BN, UW), lambda i: (i, 0)),
    out_shape=jax.ShapeDtypeStruct((N, UW), jnp.float32),
)


def _node_update_body(l, ps_ref, lid_ref, h_ref, x_ref, gw_ref, gb_ref,
                      mw_ref, wih_ref, whh_ref, bih_ref, bhh_ref,
                      ho_ref, uo_ref):
    psum = ps_ref[0] + ps_ref[1]
    denom = jnp.maximum(psum[:, VHS:VHS + 1], 1.0)
    ps = psum[:, :VHS] / denom

    h = h_ref[...]
    x = x_ref[...]
    hs1 = jnp.concatenate([h, x], axis=1)

    gi = _dot_t(ps, wih_ref[...]) + bih_ref[...]
    gh = _dot_t(hs1, whh_ref[...]) + bhh_ref[...]
    r = jax.nn.sigmoid(gi[:, :H] + gh[:, :H])
    z = jax.nn.sigmoid(gi[:, H:2 * H] + gh[:, H:2 * H])
    n = jnp.tanh(gi[:, 2 * H:] + r * gh[:, 2 * H:])
    new_h = (1.0 - z) * n + z * hs1

    mask = lid_ref[...] == l
    h_new = jnp.where(mask, new_h[:, :VHS], h)
    ho_ref[...] = h_new

    hs1n = jnp.concatenate([h_new, x], axis=1)
    uo_ref[...] = _u_rows(hs1n, gw_ref[...], gb_ref[...], mw_ref[...])


def _node_update(l):
    return pl.pallas_call(
        functools.partial(_node_update_body, l),
        grid=(GRID,),
        in_specs=[
            pl.BlockSpec((2, BN, UW), lambda i: (0, i, 0)),
            pl.BlockSpec((BN, 1), lambda i: (i, 0)),
            pl.BlockSpec((BN, VHS), lambda i: (i, 0)),
            pl.BlockSpec((BN, NVT), lambda i: (i, 0)),
            _full((VHS, H)),
            _full((1, VHS)),
            _full((VHS, H)),
            _full((3 * H, VHS)),
            _full((3 * H, H)),
            _full((1, 3 * H)),
            _full((1, 3 * H)),
        ],
        out_specs=[
            pl.BlockSpec((BN, VHS), lambda i: (i, 0)),
            pl.BlockSpec((BN, UW), lambda i: (i, 0)),
        ],
        out_shape=[
            jax.ShapeDtypeStruct((N, VHS), jnp.float32),
            jax.ShapeDtypeStruct((N, UW), jnp.float32),
        ],
    )


def kernel(x, edge_index, layer_ids, n_clause, transfer_to_device,
           w_init, gate_W, gate_b, map_W, W_ih, W_hh, b_ih, b_hh):
    src = edge_index[0]
    dst = edge_index[1]

    marks = jnp.arange(1, NL + 1, dtype=jnp.int32)
    nb = jnp.searchsorted(layer_ids, marks).astype(jnp.int32)
    lb = layer_ids[dst]
    m1 = (lb == 1).astype(jnp.int32)
    m2 = (lb == 2).astype(jnp.int32)
    m3 = (lb == 3).astype(jnp.int32)
    m0 = 1 - m1 - m2 - m3
    cs1 = jnp.cumsum(m1)
    cs2 = jnp.cumsum(m2)
    cs3 = jnp.cumsum(m3)
    cs0 = jnp.cumsum(m0)
    c1, c2, c3 = cs1[-1], cs2[-1], cs3[-1]

    def up(v):
        return ((v + 1023) // 1024) * 1024

    o2 = up(c1)
    o3 = o2 + up(c2)
    o0 = o3 + up(c3)
    pos = (m1 * (cs1 - 1) + m2 * (o2 + cs2 - 1) + m3 * (o3 + cs3 - 1)
           + m0 * (o0 + cs0 - 1))
    p = jnp.arange(EPAD, dtype=jnp.int32)
    src_s = (p % N).at[pos].set(src, unique_indices=True,
                                mode="promise_in_bounds").reshape(EROWS, CH)
    dst_s = (N + p % 128).at[pos].set(dst, unique_indices=True,
                                      mode="promise_in_bounds").reshape(EROWS, CH)

    seg_start = jnp.stack([jnp.int32(0), o2, o3])
    seg_cnt = jnp.stack([c1, c2, c3])

    def params_for(l):
        sc0 = seg_start[l - 1] // CH
        nsup = (seg_cnt[l - 1] + 1023) // 1024
        n_lo = nb[l - 1]
        n_hi = nb[l]
        a0 = (n_lo // 128) * 128
        c_ch = (n_hi - a0 + 127) // 128
        vals = jnp.stack([sc0, nsup, a0, c_ch]).astype(jnp.int32)
        return jnp.zeros((16,), jnp.int32).at[:4].set(vals)

    params = {l: params_for(l) for l in range(1, NL)}

    zeros128 = jnp.zeros((128, UW), jnp.float32)
    lid2 = layer_ids[:, None]
    gb2 = gate_b[None, :]
    bih2 = b_ih[None, :]
    bhh2 = b_hh[None, :]

    h = jnp.tile(w_init[None, :], (N, 1))
    u = _u_init(h, x, gate_W, gb2, map_W)

    for _round in range(NROUNDS):
        for l in range(1, NL):
            ps = _edge_aggregate(u, src_s, dst_s, params[l], zeros128)
            h, u = _node_update(l)(ps, lid2, h, x, gate_W, gb2, map_W,
                                   W_ih, W_hh, bih2, bhh2)
    return h


# speedup vs baseline: 2.8060x; 2.8060x over previous
import functools

import jax
import jax.numpy as jnp
from jax import lax
from jax.experimental import pallas as pl
from jax.experimental.pallas import tpu as pltpu
from jax.experimental.pallas import tpu_sc as plsc

N = 10000
E = 160000
NVT = 6
VHS = 100
H = VHS + NVT
NL = 4
NROUNDS = 2

UW = 128
NP = 10240
ROWS_PER_SUB = NP // 16
NWORK = 32
CH = 128
EROWS = 1328
EPAD = EROWS * 128
BN = 1000
GRID = N // BN

_C11 = (((1,), (1,)), ((), ()))


def _dot_t(a, b):
    return lax.dot_general(a, b, _C11, precision=lax.Precision.HIGHEST,
                           preferred_element_type=jnp.float32)


def _u_rows(hs1, gw, gb, mw):
    g = jax.nn.sigmoid(_dot_t(hs1, gw) + gb)
    p = jnp.tanh(_dot_t(hs1, mw))
    u = g * p
    rows = u.shape[0]
    return jnp.concatenate(
        [u, jnp.ones((rows, 1), jnp.float32),
         jnp.zeros((rows, UW - VHS - 1), jnp.float32)], axis=1)


_SC_MESH = plsc.VectorSubcoreMesh(core_axis_name="c", subcore_axis_name="s")


NBUF = 2
KMAX = 40


@functools.partial(
    pl.kernel,
    mesh=_SC_MESH,
    out_type=jax.ShapeDtypeStruct((2 * NP, UW), jnp.float32),
    scratch_types=[
        pltpu.VMEM((16,), jnp.int32),
        pltpu.VMEM((KMAX, CH), jnp.int32),
        pltpu.VMEM((KMAX, CH), jnp.int32),
        pltpu.VMEM((NBUF, CH, UW), jnp.float32),
        pltpu.VMEM_SHARED((NP, UW), jnp.float32),
        [pltpu.SemaphoreType.DMA] * NBUF,
        [pltpu.SemaphoreType.DMA] * NBUF,
    ],
)
def _edge_aggregate_sc(u_hbm, src_hbm, dst_hbm, params_hbm, zeros_hbm, out_hbm,
                       params_v, src_iv, dst_iv, rows_v, ps_sh, gsem, ssem):
    c = lax.axis_index("c")
    s = lax.axis_index("s")
    w = c * 16 + s

    pltpu.sync_copy(params_hbm, params_v)
    pv = params_v[...]

    sc0 = pv[0]
    nsup = pv[1]
    a0 = pv[2]
    c_ch = pv[3]

    zc = (c_ch - s + 15) // 16
    def zbody(k, carry):
        r0 = pl.multiple_of(a0 + (s + 16 * k) * 128, 128)
        pltpu.sync_copy(zeros_hbm, ps_sh.at[pl.ds(r0, 128)])
        return carry
    lax.fori_loop(0, zc, zbody, 0)

    k0 = pl.multiple_of(sc0 + 8 * ((nsup * w) // NWORK), 8)
    k1 = sc0 + 8 * ((nsup * (w + 1)) // NWORK)
    nw = k1 - k0
    pltpu.sync_copy(src_hbm.at[pl.ds(k0, KMAX)], src_iv)
    pltpu.sync_copy(dst_hbm.at[pl.ds(k0, KMAX)], dst_iv)
    plsc.subcore_barrier()

    def fire_gather(jj, b):
        pltpu.async_copy(u_hbm.at[src_iv.at[jj]], rows_v.at[b], gsem[b])

    def wait_gather(b):
        pltpu.make_async_copy(zeros_hbm, rows_v.at[b], gsem[b]).wait()

    def fire_scatter(jj, b):
        pltpu.async_copy(rows_v.at[b], ps_sh.at[dst_iv.at[jj]], ssem[b],
                         add=True)

    def wait_scatter(b):
        pltpu.make_async_copy(zeros_hbm, rows_v.at[b], ssem[b]).wait()

    n_grp = (nw + NBUF - 1) // NBUF
    def group(i, carry):
        for b in range(NBUF):
            jj = i * NBUF + b
            @pl.when((jj < nw) & (i > 0))
            def _():
                wait_scatter(b)
            @pl.when(jj < nw)
            def _():
                fire_gather(jj, b)
        for b in range(NBUF):
            jj = i * NBUF + b
            @pl.when(jj < nw)
            def _():
                wait_gather(b)
                fire_scatter(jj, b)
        return carry
    lax.fori_loop(0, n_grp, group, 0)
    for b in range(NBUF):
        @pl.when(b < nw)
        def _():
            wait_scatter(b)
    plsc.subcore_barrier()

    def obody(k, carry):
        r0 = pl.multiple_of(a0 + (s + 16 * k) * 128, 128)
        pltpu.sync_copy(ps_sh.at[pl.ds(r0, 128)],
                        out_hbm.at[pl.ds(c * NP + r0, 128)])
        return carry
    lax.fori_loop(0, zc, obody, 0)


def _edge_aggregate(u, src_s, dst_s, params, zeros128):
    return _edge_aggregate_sc(u, src_s, dst_s, params, zeros128).reshape(
        2, NP, UW)


def _full(shape):
    return pl.BlockSpec(shape, lambda i: tuple(0 for _ in shape))


def _u_init_body(h_ref, x_ref, gw_ref, gb_ref, mw_ref, u_ref):
    hs1 = jnp.concatenate([h_ref[...], x_ref[...]], axis=1)
    u_ref[...] = _u_rows(hs1, gw_ref[...], gb_ref[...], mw_ref[...])


_u_init = pl.pallas_call(
    _u_init_body,
    grid=(GRID,),
    in_specs=[
        pl.BlockSpec((BN, VHS), lambda i: (i, 0)),
        pl.BlockSpec((BN, NVT), lambda i: (i, 0)),
        _full((VHS, H)),
        _full((1, VHS)),
        _full((VHS, H)),
    ],
    out_specs=pl.BlockSpec((BN, UW), lambda i: (i, 0)),
    out_shape=jax.ShapeDtypeStruct((N, UW), jnp.float32),
)


def _node_update_body(l, ps_ref, lid_ref, h_ref, x_ref, gw_ref, gb_ref,
                      mw_ref, wih_ref, whh_ref, bih_ref, bhh_ref,
                      ho_ref, uo_ref):
    psum = ps_ref[0] + ps_ref[1]
    denom = jnp.maximum(psum[:, VHS:VHS + 1], 1.0)
    ps = psum[:, :VHS] / denom

    h = h_ref[...]
    x = x_ref[...]
    hs1 = jnp.concatenate([h, x], axis=1)

    gi = _dot_t(ps, wih_ref[...]) + bih_ref[...]
    gh = _dot_t(hs1, whh_ref[...]) + bhh_ref[...]
    r = jax.nn.sigmoid(gi[:, :H] + gh[:, :H])
    z = jax.nn.sigmoid(gi[:, H:2 * H] + gh[:, H:2 * H])
    n = jnp.tanh(gi[:, 2 * H:] + r * gh[:, 2 * H:])
    new_h = (1.0 - z) * n + z * hs1

    mask = lid_ref[...] == l
    h_new = jnp.where(mask, new_h[:, :VHS], h)
    ho_ref[...] = h_new

    hs1n = jnp.concatenate([h_new, x], axis=1)
    uo_ref[...] = _u_rows(hs1n, gw_ref[...], gb_ref[...], mw_ref[...])


def _node_update(l):
    return pl.pallas_call(
        functools.partial(_node_update_body, l),
        grid=(GRID,),
        in_specs=[
            pl.BlockSpec((2, BN, UW), lambda i: (0, i, 0)),
            pl.BlockSpec((BN, 1), lambda i: (i, 0)),
            pl.BlockSpec((BN, VHS), lambda i: (i, 0)),
            pl.BlockSpec((BN, NVT), lambda i: (i, 0)),
            _full((VHS, H)),
            _full((1, VHS)),
            _full((VHS, H)),
            _full((3 * H, VHS)),
            _full((3 * H, H)),
            _full((1, 3 * H)),
            _full((1, 3 * H)),
        ],
        out_specs=[
            pl.BlockSpec((BN, VHS), lambda i: (i, 0)),
            pl.BlockSpec((BN, UW), lambda i: (i, 0)),
        ],
        out_shape=[
            jax.ShapeDtypeStruct((N, VHS), jnp.float32),
            jax.ShapeDtypeStruct((N, UW), jnp.float32),
        ],
    )


def kernel(x, edge_index, layer_ids, n_clause, transfer_to_device,
           w_init, gate_W, gate_b, map_W, W_ih, W_hh, b_ih, b_hh):
    src = edge_index[0]
    dst = edge_index[1]

    perm = jnp.argsort(dst)
    srcs = src[perm]
    dsts = dst[perm]
    marks = jnp.arange(1, NL + 1, dtype=jnp.int32)
    nb = jnp.searchsorted(layer_ids, marks).astype(jnp.int32)
    b = jnp.searchsorted(dsts, nb[:NL - 1]).astype(jnp.int32)
    b0, b1, b2 = b[0], b[1], b[2]
    c1, c2, c3 = b1 - b0, b2 - b1, E - b2

    def up(v):
        return ((v + 1023) // 1024) * 1024

    o2 = up(c1)
    o3 = o2 + up(c2)
    o0 = o3 + up(c3)
    start_pad = jnp.stack([jnp.int32(0), o2, o3, o0])
    start_raw = jnp.stack([b0, b1, b2, jnp.int32(0)])
    end_raw = jnp.stack([b1, b2, jnp.int32(E), b0])
    p = jnp.arange(EPAD, dtype=jnp.int32)
    li = ((p >= o2).astype(jnp.int32) + (p >= o3) + (p >= o0))
    i = p - start_pad[li] + start_raw[li]
    valid = i < end_raw[li]
    i_c = jnp.minimum(i, E - 1)
    src_s = jnp.where(valid, srcs[i_c], p % N).reshape(EROWS, CH)
    dst_s = jnp.where(valid, dsts[i_c], N + p % 128).reshape(EROWS, CH)

    seg_start = jnp.stack([jnp.int32(0), o2, o3])
    seg_cnt = jnp.stack([c1, c2, c3])

    def params_for(l):
        sc0 = seg_start[l - 1] // CH
        nsup = (seg_cnt[l - 1] + 1023) // 1024
        n_lo = nb[l - 1]
        n_hi = nb[l]
        a0 = (n_lo // 128) * 128
        c_ch = (n_hi - a0 + 127) // 128
        vals = jnp.stack([sc0, nsup, a0, c_ch]).astype(jnp.int32)
        return jnp.zeros((16,), jnp.int32).at[:4].set(vals)

    params = {l: params_for(l) for l in range(1, NL)}

    zeros128 = jnp.zeros((128, UW), jnp.float32)
    lid2 = layer_ids[:, None]
    gb2 = gate_b[None, :]
    bih2 = b_ih[None, :]
    bhh2 = b_hh[None, :]

    h = jnp.tile(w_init[None, :], (N, 1))
    u = _u_init(h, x, gate_W, gb2, map_W)

    for _round in range(NROUNDS):
        for l in range(1, NL):
            ps = _edge_aggregate(u, src_s, dst_s, params[l], zeros128)
            h, u = _node_update(l)(ps, lid2, h, x, gate_W, gb2, map_W,
                                   W_ih, W_hh, bih2, bhh2)
    return h


# speedup vs baseline: 3.4065x; 1.2140x over previous
import functools

import jax
import jax.numpy as jnp
from jax import lax
from jax.experimental import pallas as pl
from jax.experimental.pallas import tpu as pltpu
from jax.experimental.pallas import tpu_sc as plsc

N = 10000
E = 160000
NVT = 6
VHS = 100
H = VHS + NVT
NL = 4
NROUNDS = 2

UW = 128
NP = 10240
ROWS_PER_SUB = NP // 16
NWORK = 32
CH = 128
EROWS = 1328
EPAD = EROWS * 128
BN = 1000
GRID = N // BN

_C11 = (((1,), (1,)), ((), ()))


def _dot_t(a, b):
    return lax.dot_general(a, b, _C11, precision=lax.Precision.HIGHEST,
                           preferred_element_type=jnp.float32)


def _u_rows(hs1, gw, gb, mw):
    g = jax.nn.sigmoid(_dot_t(hs1, gw) + gb)
    p = jnp.tanh(_dot_t(hs1, mw))
    u = g * p
    rows = u.shape[0]
    return jnp.concatenate(
        [u, jnp.ones((rows, 1), jnp.float32),
         jnp.zeros((rows, UW - VHS - 1), jnp.float32)], axis=1)


_SC_MESH = plsc.VectorSubcoreMesh(core_axis_name="c", subcore_axis_name="s")


NBUF = 2
KMAX = 40


@functools.partial(
    pl.kernel,
    mesh=_SC_MESH,
    out_type=jax.ShapeDtypeStruct((2 * NP, UW), jnp.float32),
    scratch_types=[
        pltpu.VMEM((16,), jnp.int32),
        pltpu.VMEM((KMAX, CH), jnp.int32),
        pltpu.VMEM((KMAX, CH), jnp.int32),
        pltpu.VMEM((NBUF, CH, UW), jnp.float32),
        pltpu.VMEM_SHARED((NP, UW), jnp.float32),
        [pltpu.SemaphoreType.DMA] * NBUF,
        [pltpu.SemaphoreType.DMA] * NBUF,
    ],
)
def _edge_aggregate_sc(u_hbm, src_hbm, dst_hbm, params_hbm, zeros_hbm, out_hbm,
                       params_v, src_iv, dst_iv, rows_v, ps_sh, gsem, ssem):
    c = lax.axis_index("c")
    s = lax.axis_index("s")
    w = c * 16 + s

    pltpu.sync_copy(params_hbm, params_v)
    pv = params_v[...]

    sc0 = pv[0]
    nsup = pv[1]
    a0 = pv[2]
    c_ch = pv[3]

    zc = (c_ch - s + 15) // 16
    def zbody(k, carry):
        r0 = pl.multiple_of(a0 + (s + 16 * k) * 128, 128)
        pltpu.sync_copy(zeros_hbm, ps_sh.at[pl.ds(r0, 128)])
        return carry
    lax.fori_loop(0, zc, zbody, 0)

    k0 = pl.multiple_of(sc0 + 8 * ((nsup * w) // NWORK), 8)
    k1 = sc0 + 8 * ((nsup * (w + 1)) // NWORK)
    nw = k1 - k0
    pltpu.sync_copy(src_hbm.at[pl.ds(k0, KMAX)], src_iv)
    pltpu.sync_copy(dst_hbm.at[pl.ds(k0, KMAX)], dst_iv)
    plsc.subcore_barrier()

    def fire_gather(jj, b):
        pltpu.async_copy(u_hbm.at[src_iv.at[jj]], rows_v.at[b], gsem[b])

    def wait_gather(b):
        pltpu.make_async_copy(zeros_hbm, rows_v.at[b], gsem[b]).wait()

    def fire_scatter(jj, b):
        pltpu.async_copy(rows_v.at[b], ps_sh.at[dst_iv.at[jj]], ssem[b],
                         add=True)

    def wait_scatter(b):
        pltpu.make_async_copy(zeros_hbm, rows_v.at[b], ssem[b]).wait()

    n_grp = (nw + NBUF - 1) // NBUF
    def group(i, carry):
        for b in range(NBUF):
            jj = i * NBUF + b
            @pl.when((jj < nw) & (i > 0))
            def _():
                wait_scatter(b)
            @pl.when(jj < nw)
            def _():
                fire_gather(jj, b)
        for b in range(NBUF):
            jj = i * NBUF + b
            @pl.when(jj < nw)
            def _():
                wait_gather(b)
                fire_scatter(jj, b)
        return carry
    lax.fori_loop(0, n_grp, group, 0)
    for b in range(NBUF):
        @pl.when(b < nw)
        def _():
            wait_scatter(b)
    plsc.subcore_barrier()

    def obody(k, carry):
        r0 = pl.multiple_of(a0 + (s + 16 * k) * 128, 128)
        pltpu.sync_copy(ps_sh.at[pl.ds(r0, 128)],
                        out_hbm.at[pl.ds(c * NP + r0, 128)])
        return carry
    lax.fori_loop(0, zc, obody, 0)


def _edge_aggregate(u, src_s, dst_s, params, zeros128):
    return _edge_aggregate_sc(u, src_s, dst_s, params, zeros128).reshape(
        2, NP, UW)


def _full(shape):
    return pl.BlockSpec(shape, lambda i: tuple(0 for _ in shape))


def _u_init_body(h_ref, x_ref, gw_ref, gb_ref, mw_ref, u_ref):
    hs1 = jnp.concatenate([h_ref[...], x_ref[...]], axis=1)
    u_ref[...] = _u_rows(hs1, gw_ref[...], gb_ref[...], mw_ref[...])


_u_init = pl.pallas_call(
    _u_init_body,
    grid=(GRID,),
    in_specs=[
        pl.BlockSpec((BN, VHS), lambda i: (i, 0)),
        pl.BlockSpec((BN, NVT), lambda i: (i, 0)),
        _full((VHS, H)),
        _full((1, VHS)),
        _full((VHS, H)),
    ],
    out_specs=pl.BlockSpec((BN, UW), lambda i: (i, 0)),
    out_shape=jax.ShapeDtypeStruct((N, UW), jnp.float32),
)


def _node_update_body(l, ps_ref, lid_ref, h_ref, x_ref, u_ref, gw_ref, gb_ref,
                      mw_ref, wih_ref, whh_ref, bih_ref, bhh_ref,
                      ho_ref, uo_ref):
    mask = lid_ref[...] == l
    active = jnp.any(mask)

    @pl.when(active)
    def _():
        psum = ps_ref[0] + ps_ref[1]
        denom = jnp.maximum(psum[:, VHS:VHS + 1], 1.0)
        ps = psum[:, :VHS] / denom

        h = h_ref[...]
        x = x_ref[...]
        hs1 = jnp.concatenate([h, x], axis=1)

        gi = _dot_t(ps, wih_ref[...]) + bih_ref[...]
        gh = _dot_t(hs1, whh_ref[...]) + bhh_ref[...]
        r = jax.nn.sigmoid(gi[:, :H] + gh[:, :H])
        z = jax.nn.sigmoid(gi[:, H:2 * H] + gh[:, H:2 * H])
        n = jnp.tanh(gi[:, 2 * H:] + r * gh[:, 2 * H:])
        new_h = (1.0 - z) * n + z * hs1

        h_new = jnp.where(mask, new_h[:, :VHS], h)
        ho_ref[...] = h_new
        hs1n = jnp.concatenate([h_new, x], axis=1)
        uo_ref[...] = _u_rows(hs1n, gw_ref[...], gb_ref[...], mw_ref[...])

    @pl.when(jnp.logical_not(active))
    def _():
        ho_ref[...] = h_ref[...]
        uo_ref[...] = u_ref[...]


def _node_update(l):
    return pl.pallas_call(
        functools.partial(_node_update_body, l),
        grid=(GRID,),
        in_specs=[
            pl.BlockSpec((2, BN, UW), lambda i: (0, i, 0)),
            pl.BlockSpec((BN, 1), lambda i: (i, 0)),
            pl.BlockSpec((BN, VHS), lambda i: (i, 0)),
            pl.BlockSpec((BN, NVT), lambda i: (i, 0)),
            pl.BlockSpec((BN, UW), lambda i: (i, 0)),
            _full((VHS, H)),
            _full((1, VHS)),
            _full((VHS, H)),
            _full((3 * H, VHS)),
            _full((3 * H, H)),
            _full((1, 3 * H)),
            _full((1, 3 * H)),
        ],
        out_specs=[
            pl.BlockSpec((BN, VHS), lambda i: (i, 0)),
            pl.BlockSpec((BN, UW), lambda i: (i, 0)),
        ],
        out_shape=[
            jax.ShapeDtypeStruct((N, VHS), jnp.float32),
            jax.ShapeDtypeStruct((N, UW), jnp.float32),
        ],
    )


def kernel(x, edge_index, layer_ids, n_clause, transfer_to_device,
           w_init, gate_W, gate_b, map_W, W_ih, W_hh, b_ih, b_hh):
    src = edge_index[0]
    dst = edge_index[1]

    perm = jnp.argsort(dst)
    srcs = src[perm]
    dsts = dst[perm]
    marks = jnp.arange(1, NL + 1, dtype=jnp.int32)
    nb = jnp.searchsorted(layer_ids, marks).astype(jnp.int32)
    b = jnp.searchsorted(dsts, nb[:NL - 1]).astype(jnp.int32)
    b0, b1, b2 = b[0], b[1], b[2]
    c1, c2, c3 = b1 - b0, b2 - b1, E - b2

    def up(v):
        return ((v + 1023) // 1024) * 1024

    o2 = up(c1)
    o3 = o2 + up(c2)
    o0 = o3 + up(c3)
    start_pad = jnp.stack([jnp.int32(0), o2, o3, o0])
    start_raw = jnp.stack([b0, b1, b2, jnp.int32(0)])
    end_raw = jnp.stack([b1, b2, jnp.int32(E), b0])
    p = jnp.arange(EPAD, dtype=jnp.int32)
    li = ((p >= o2).astype(jnp.int32) + (p >= o3) + (p >= o0))
    i = p - start_pad[li] + start_raw[li]
    valid = i < end_raw[li]
    i_c = jnp.minimum(i, E - 1)
    src_s = jnp.where(valid, srcs[i_c], p % N).reshape(EROWS, CH)
    dst_s = jnp.where(valid, dsts[i_c], N + p % 128).reshape(EROWS, CH)

    seg_start = jnp.stack([jnp.int32(0), o2, o3])
    seg_cnt = jnp.stack([c1, c2, c3])

    def params_for(l):
        sc0 = seg_start[l - 1] // CH
        nsup = (seg_cnt[l - 1] + 1023) // 1024
        n_lo = nb[l - 1]
        n_hi = nb[l]
        a0 = (n_lo // 128) * 128
        c_ch = (n_hi - a0 + 127) // 128
        vals = jnp.stack([sc0, nsup, a0, c_ch]).astype(jnp.int32)
        return jnp.zeros((16,), jnp.int32).at[:4].set(vals)

    params = {l: params_for(l) for l in range(1, NL)}

    zeros128 = jnp.zeros((128, UW), jnp.float32)
    lid2 = layer_ids[:, None]
    gb2 = gate_b[None, :]
    bih2 = b_ih[None, :]
    bhh2 = b_hh[None, :]

    h = jnp.tile(w_init[None, :], (N, 1))
    u = _u_init(h, x, gate_W, gb2, map_W)

    for _round in range(NROUNDS):
        for l in range(1, NL):
            ps = _edge_aggregate(u, src_s, dst_s, params[l], zeros128)
            h, u = _node_update(l)(ps, lid2, h, x, u, gate_W, gb2, map_W,
                                   W_ih, W_hh, bih2, bhh2)
    return h


# speedup vs baseline: 3.6557x; 1.0732x over previous
import functools

import jax
import jax.numpy as jnp
from jax import lax
from jax.experimental import pallas as pl
from jax.experimental.pallas import tpu as pltpu
from jax.experimental.pallas import tpu_sc as plsc

N = 10000
E = 160000
NVT = 6
VHS = 100
H = VHS + NVT
NL = 4
NROUNDS = 2

UW = 128
NP = 10240
ROWS_PER_SUB = NP // 16
NWORK = 32
CH = 128
EROWS = 1328
EPAD = EROWS * 128
BN = 1000
GRID = N // BN

_C11 = (((1,), (1,)), ((), ()))


def _dot_t(a, b):
    return lax.dot_general(a, b, _C11, precision=lax.Precision.HIGHEST,
                           preferred_element_type=jnp.float32)


def _u_rows(hs1, gw, gb, mw):
    g = jax.nn.sigmoid(_dot_t(hs1, gw) + gb)
    p = jnp.tanh(_dot_t(hs1, mw))
    u = g * p
    rows = u.shape[0]
    return jnp.concatenate(
        [u, jnp.ones((rows, 1), jnp.float32),
         jnp.zeros((rows, UW - VHS - 1), jnp.float32)], axis=1)


_SC_MESH = plsc.VectorSubcoreMesh(core_axis_name="c", subcore_axis_name="s")


NBUF = 2
KMAX = 40


@functools.partial(
    pl.kernel,
    mesh=_SC_MESH,
    out_type=jax.ShapeDtypeStruct((2 * NP, UW), jnp.float32),
    scratch_types=[
        pltpu.VMEM((16,), jnp.int32),
        pltpu.VMEM((KMAX, CH), jnp.int32),
        pltpu.VMEM((KMAX, CH), jnp.int32),
        pltpu.VMEM((NBUF, CH, UW), jnp.float32),
        pltpu.VMEM_SHARED((NP, UW), jnp.float32),
        [pltpu.SemaphoreType.DMA] * NBUF,
        [pltpu.SemaphoreType.DMA] * NBUF,
    ],
)
def _edge_aggregate_sc(u_hbm, src_hbm, dst_hbm, params_hbm, zeros_hbm, out_hbm,
                       params_v, src_iv, dst_iv, rows_v, ps_sh, gsem, ssem):
    c = lax.axis_index("c")
    s = lax.axis_index("s")
    w = c * 16 + s

    pltpu.sync_copy(params_hbm, params_v)
    pv = params_v[...]

    sc0 = pv[0]
    nsup = pv[1]
    a0 = pv[2]
    c_ch = pv[3]

    zc = (c_ch - s + 15) // 16
    def zbody(k, carry):
        r0 = pl.multiple_of(a0 + (s + 16 * k) * 128, 128)
        pltpu.sync_copy(zeros_hbm, ps_sh.at[pl.ds(r0, 128)])
        return carry
    lax.fori_loop(0, zc, zbody, 0)

    k0 = pl.multiple_of(sc0 + 8 * ((nsup * w) // NWORK), 8)
    k1 = sc0 + 8 * ((nsup * (w + 1)) // NWORK)
    nw = k1 - k0
    pltpu.sync_copy(src_hbm.at[pl.ds(k0, KMAX)], src_iv)
    pltpu.sync_copy(dst_hbm.at[pl.ds(k0, KMAX)], dst_iv)
    plsc.subcore_barrier()

    def fire_gather(jj, b):
        pltpu.async_copy(u_hbm.at[src_iv.at[jj]], rows_v.at[b], gsem[b])

    def wait_gather(b):
        pltpu.make_async_copy(zeros_hbm, rows_v.at[b], gsem[b]).wait()

    def fire_scatter(jj, b):
        pltpu.async_copy(rows_v.at[b], ps_sh.at[dst_iv.at[jj]], ssem[b],
                         add=True)

    def wait_scatter(b):
        pltpu.make_async_copy(zeros_hbm, rows_v.at[b], ssem[b]).wait()

    n_grp = (nw + NBUF - 1) // NBUF
    def group(i, carry):
        for b in range(NBUF):
            jj = i * NBUF + b
            @pl.when((jj < nw) & (i > 0))
            def _():
                wait_scatter(b)
            @pl.when(jj < nw)
            def _():
                fire_gather(jj, b)
        for b in range(NBUF):
            jj = i * NBUF + b
            @pl.when(jj < nw)
            def _():
                wait_gather(b)
                fire_scatter(jj, b)
        return carry
    lax.fori_loop(0, n_grp, group, 0)
    for b in range(NBUF):
        @pl.when(b < nw)
        def _():
            wait_scatter(b)
    plsc.subcore_barrier()

    def obody(k, carry):
        r0 = pl.multiple_of(a0 + (s + 16 * k) * 128, 128)
        pltpu.sync_copy(ps_sh.at[pl.ds(r0, 128)],
                        out_hbm.at[pl.ds(c * NP + r0, 128)])
        return carry
    lax.fori_loop(0, zc, obody, 0)


def _edge_aggregate(u, src_s, dst_s, params, zeros128):
    return _edge_aggregate_sc(u, src_s, dst_s, params, zeros128).reshape(
        2, NP, UW)


def _full(shape):
    return pl.BlockSpec(shape, lambda i: tuple(0 for _ in shape))


def _u_init_body(h_ref, x_ref, gw_ref, gb_ref, mw_ref, u_ref):
    hs1 = jnp.concatenate([h_ref[...], x_ref[...]], axis=1)
    u_ref[...] = _u_rows(hs1, gw_ref[...], gb_ref[...], mw_ref[...])


_u_init = pl.pallas_call(
    _u_init_body,
    grid=(GRID,),
    in_specs=[
        pl.BlockSpec((BN, VHS), lambda i: (i, 0)),
        pl.BlockSpec((BN, NVT), lambda i: (i, 0)),
        _full((VHS, H)),
        _full((1, VHS)),
        _full((VHS, H)),
    ],
    out_specs=pl.BlockSpec((BN, UW), lambda i: (i, 0)),
    out_shape=jax.ShapeDtypeStruct((N, UW), jnp.float32),
)


def _node_update_body(l, ps_ref, lid_ref, h_ref, x_ref, u_ref, gw_ref, gb_ref,
                      mw_ref, wih_ref, whh_ref, bih_ref, bhh_ref,
                      ho_ref, uo_ref):
    mask = lid_ref[...] == l
    active = jnp.any(mask)

    @pl.when(active)
    def _():
        psum = ps_ref[0] + ps_ref[1]
        denom = jnp.maximum(psum[:, VHS:VHS + 1], 1.0)
        ps = psum[:, :VHS] / denom

        h = h_ref[...]
        x = x_ref[...]
        hs1 = jnp.concatenate([h, x], axis=1)

        gi = _dot_t(ps, wih_ref[...]) + bih_ref[...]
        gh = _dot_t(hs1, whh_ref[...]) + bhh_ref[...]
        r = jax.nn.sigmoid(gi[:, :H] + gh[:, :H])
        z = jax.nn.sigmoid(gi[:, H:2 * H] + gh[:, H:2 * H])
        n = jnp.tanh(gi[:, 2 * H:] + r * gh[:, 2 * H:])
        new_h = (1.0 - z) * n + z * hs1

        h_new = jnp.where(mask, new_h[:, :VHS], h)
        ho_ref[...] = h_new
        hs1n = jnp.concatenate([h_new, x], axis=1)
        uo_ref[...] = _u_rows(hs1n, gw_ref[...], gb_ref[...], mw_ref[...])

    @pl.when(jnp.logical_not(active))
    def _():
        ho_ref[...] = h_ref[...]
        uo_ref[...] = u_ref[...]


def _node_update(l):
    return pl.pallas_call(
        functools.partial(_node_update_body, l),
        grid=(GRID,),
        in_specs=[
            pl.BlockSpec((2, BN, UW), lambda i: (0, i, 0)),
            pl.BlockSpec((BN, 1), lambda i: (i, 0)),
            pl.BlockSpec((BN, VHS), lambda i: (i, 0)),
            pl.BlockSpec((BN, NVT), lambda i: (i, 0)),
            pl.BlockSpec((BN, UW), lambda i: (i, 0)),
            _full((VHS, H)),
            _full((1, VHS)),
            _full((VHS, H)),
            _full((3 * H, VHS)),
            _full((3 * H, H)),
            _full((1, 3 * H)),
            _full((1, 3 * H)),
        ],
        out_specs=[
            pl.BlockSpec((BN, VHS), lambda i: (i, 0)),
            pl.BlockSpec((BN, UW), lambda i: (i, 0)),
        ],
        out_shape=[
            jax.ShapeDtypeStruct((N, VHS), jnp.float32),
            jax.ShapeDtypeStruct((N, UW), jnp.float32),
        ],
    )


def kernel(x, edge_index, layer_ids, n_clause, transfer_to_device,
           w_init, gate_W, gate_b, map_W, W_ih, W_hh, b_ih, b_hh):
    src = edge_index[0]
    dst = edge_index[1]

    marks = jnp.arange(1, NL + 1, dtype=jnp.int32)
    nb = jnp.searchsorted(layer_ids, marks).astype(jnp.int32)
    p = jnp.arange(EPAD - E, dtype=jnp.int32)
    src_s = jnp.concatenate([src, p % N]).reshape(EROWS, CH)
    dst_s = jnp.concatenate([dst, N + p % 128]).reshape(EROWS, CH)
    nsup_all = (E + 1023) // 1024

    def params_for(l):
        n_lo = nb[l - 1]
        n_hi = nb[l]
        a0 = (n_lo // 128) * 128
        c_ch = (n_hi - a0 + 127) // 128
        vals = jnp.stack([jnp.int32(0), jnp.int32(nsup_all), a0,
                          c_ch]).astype(jnp.int32)
        return jnp.zeros((16,), jnp.int32).at[:4].set(vals)

    params = {l: params_for(l) for l in range(1, NL)}

    zeros128 = jnp.zeros((128, UW), jnp.float32)
    lid2 = layer_ids[:, None]
    gb2 = gate_b[None, :]
    bih2 = b_ih[None, :]
    bhh2 = b_hh[None, :]

    h = jnp.tile(w_init[None, :], (N, 1))
    u = _u_init(h, x, gate_W, gb2, map_W)

    for _round in range(NROUNDS):
        for l in range(1, NL):
            ps = _edge_aggregate(u, src_s, dst_s, params[l], zeros128)
            h, u = _node_update(l)(ps, lid2, h, x, u, gate_W, gb2, map_W,
                                   W_ih, W_hh, bih2, bhh2)
    return h
